# Initial kernel scaffold; baseline (speedup 1.0000x reference)
import functools

import jax
import jax.numpy as jnp
from jax import lax
from jax.experimental import pallas as pl
from jax.experimental.pallas import tpu as pltpu
from jax.experimental.pallas import tpu_sc as plsc

NC = 2
NS = 16
L = 16
NW = NC * NS

Q = 2048
R = 32768
QPW = Q // NW
QT = 4
NQT = QPW // QT
RV = R // L
RU = 4
NRI = RV // RU

_mesh = plsc.VectorSubcoreMesh(
    core_axis_name="c", subcore_axis_name="s", num_cores=NC, num_subcores=NS
)


@functools.partial(
    pl.kernel,
    out_type=jax.ShapeDtypeStruct((NW, L), jnp.float32),
    mesh=_mesh,
    scratch_types=[
        pltpu.VMEM((QPW * L,), jnp.float32),
        pltpu.VMEM((QPW * L,), jnp.float32),
        pltpu.VMEM((QPW * L,), jnp.float32),
        pltpu.VMEM((R,), jnp.float32),
        pltpu.VMEM((R,), jnp.float32),
        pltpu.VMEM((R,), jnp.float32),
        pltpu.VMEM((L,), jnp.float32),
    ],
)
def _chamfer_sc(qx_hbm, qy_hbm, qz_hbm, rx_hbm, ry_hbm, rz_hbm, out_hbm,
                qx_v, qy_v, qz_v, rx_v, ry_v, rz_v, sv):
  wid = lax.axis_index("c") * NS + lax.axis_index("s")
  qbase = wid * (QPW * L)

  pltpu.sync_copy(qx_hbm.at[pl.ds(qbase, QPW * L)], qx_v)
  pltpu.sync_copy(qy_hbm.at[pl.ds(qbase, QPW * L)], qy_v)
  pltpu.sync_copy(qz_hbm.at[pl.ds(qbase, QPW * L)], qz_v)
  pltpu.sync_copy(rx_hbm, rx_v)
  pltpu.sync_copy(ry_hbm, ry_v)
  pltpu.sync_copy(rz_hbm, rz_v)

  inf16 = jnp.full((L,), jnp.inf, dtype=jnp.float32)

  def qtile_body(qt, psum):
    qq = []
    ax = []
    ay = []
    az = []
    for t in range(QT):
      off = (qt * QT + t) * L
      qxv = qx_v[pl.ds(off, L)]
      qyv = qy_v[pl.ds(off, L)]
      qzv = qz_v[pl.ds(off, L)]
      qq.append(qxv * qxv + qyv * qyv + qzv * qzv)
      ax.append(-2.0 * qxv)
      ay.append(-2.0 * qyv)
      az.append(-2.0 * qzv)

    def rbody(i, accs):
      accs = list(accs)
      for u in range(RU):
        base = (i * RU + u) * L
        rxv = rx_v[pl.ds(base, L)]
        ryv = ry_v[pl.ds(base, L)]
        rzv = rz_v[pl.ds(base, L)]
        rrv = rxv * rxv + ryv * ryv + rzv * rzv
        for t in range(QT):
          d = rrv + qq[t] + rxv * ax[t] + ryv * ay[t] + rzv * az[t]
          accs[t] = jnp.minimum(accs[t], d)
      return tuple(accs)

    accs = lax.fori_loop(0, NRI, rbody, (inf16,) * QT)
    for t in range(QT):
      psum = psum + jnp.min(accs[t])
    return psum

  psum = lax.fori_loop(0, NQT, qtile_body, jnp.float32(0.0))
  sv[...] = lax.broadcast(psum, (L,))
  pltpu.sync_copy(sv, out_hbm.at[wid])


def kernel(query, ref):
  qrep = jnp.broadcast_to(query.T[:, :, None], (3, Q, L)).reshape(3, Q * L)
  rT = ref.T
  out = _chamfer_sc(qrep[0], qrep[1], qrep[2], rT[0], rT[1], rT[2])
  return jnp.sum(out[:, 0]) / jnp.float32(Q)


# speedup vs baseline: 3.2016x; 3.2016x over previous
import functools

import jax
import jax.numpy as jnp
from jax import lax
from jax.experimental import pallas as pl
from jax.experimental.pallas import tpu as pltpu
from jax.experimental.pallas import tpu_sc as plsc

NC = 2
NS = 16
L = 16
NW = NC * NS

Q = 2048
R = 32768
QPW = Q // NW
QT = 4
NQT = QPW // QT
NH = 2
RH = R // NH
RVH = RH // L
RU = 4
NRI = RVH // RU

_mesh = plsc.VectorSubcoreMesh(
    core_axis_name="c", subcore_axis_name="s", num_cores=NC, num_subcores=NS
)


def _round_bf16(v):
  u = lax.bitcast_convert_type(v, jnp.uint32)
  odd = lax.shift_right_logical(u, jnp.uint32(16)) & jnp.uint32(1)
  r = (u + jnp.uint32(0x7FFF) + odd) & jnp.uint32(0xFFFF0000)
  return lax.bitcast_convert_type(r, jnp.float32)


@functools.partial(
    pl.kernel,
    out_type=jax.ShapeDtypeStruct((NW, L), jnp.float32),
    mesh=_mesh,
    compiler_params=pltpu.CompilerParams(needs_layout_passes=False),
    scratch_types=[
        pltpu.VMEM((QPW * L,), jnp.float32),
        pltpu.VMEM((QPW * L,), jnp.float32),
        pltpu.VMEM((QPW * L,), jnp.float32),
        pltpu.VMEM((RH,), jnp.float32),
        pltpu.VMEM((RH,), jnp.float32),
        pltpu.VMEM((RH,), jnp.float32),
        pltpu.VMEM((RH,), jnp.float32),
        pltpu.VMEM((QPW * L,), jnp.float32),
        pltpu.VMEM((L,), jnp.float32),
    ],
)
def _chamfer_sc(qx_hbm, qy_hbm, qz_hbm, rx_hbm, ry_hbm, rz_hbm, out_hbm,
                qx_v, qy_v, qz_v, rx_v, ry_v, rz_v, rr_v, acc_v, sv):
  wid = lax.axis_index("c") * NS + lax.axis_index("s")
  qbase = wid * (QPW * L)

  pltpu.sync_copy(qx_hbm.at[pl.ds(qbase, QPW * L)], qx_v)
  pltpu.sync_copy(qy_hbm.at[pl.ds(qbase, QPW * L)], qy_v)
  pltpu.sync_copy(qz_hbm.at[pl.ds(qbase, QPW * L)], qz_v)

  inf16 = jnp.full((L,), jnp.inf, dtype=jnp.float32)

  for h in range(NH):
    pltpu.sync_copy(rx_hbm.at[pl.ds(h * RH, RH)], rx_v)
    pltpu.sync_copy(ry_hbm.at[pl.ds(h * RH, RH)], ry_v)
    pltpu.sync_copy(rz_hbm.at[pl.ds(h * RH, RH)], rz_v)

    def prologue(j, carry):
      off = j * L
      x = rx_v[pl.ds(off, L)]
      y = ry_v[pl.ds(off, L)]
      z = rz_v[pl.ds(off, L)]
      rr_v[pl.ds(off, L)] = x * x + y * y + z * z
      rx_v[pl.ds(off, L)] = _round_bf16(x)
      ry_v[pl.ds(off, L)] = _round_bf16(y)
      rz_v[pl.ds(off, L)] = _round_bf16(z)
      return carry

    lax.fori_loop(0, RVH, prologue, jnp.int32(0))

    def qtile_body(qt, carry):
      qq = []
      ax = []
      ay = []
      az = []
      for t in range(QT):
        off = (qt * QT + t) * L
        qxv = qx_v[pl.ds(off, L)]
        qyv = qy_v[pl.ds(off, L)]
        qzv = qz_v[pl.ds(off, L)]
        qq.append(qxv * qxv + qyv * qyv + qzv * qzv)
        ax.append(-2.0 * _round_bf16(qxv))
        ay.append(-2.0 * _round_bf16(qyv))
        az.append(-2.0 * _round_bf16(qzv))

      def rbody(i, accs):
        accs = list(accs)
        for u in range(RU):
          base = (i * RU + u) * L
          rxv = rx_v[pl.ds(base, L)]
          ryv = ry_v[pl.ds(base, L)]
          rzv = rz_v[pl.ds(base, L)]
          rrv = rr_v[pl.ds(base, L)]
          for t in range(QT):
            d = rrv + rxv * ax[t] + ryv * ay[t] + rzv * az[t]
            accs[t] = jnp.minimum(accs[t], d)
        return tuple(accs)

      accs = lax.fori_loop(0, NRI, rbody, (inf16,) * QT)
      for t in range(QT):
        soff = (qt * QT + t) * L
        total = accs[t] + qq[t]
        if h == 0:
          acc_v[pl.ds(soff, L)] = total
        else:
          acc_v[pl.ds(soff, L)] = jnp.minimum(acc_v[pl.ds(soff, L)], total)
      return carry

    lax.fori_loop(0, NQT, qtile_body, jnp.int32(0))

  lanes = lax.iota(jnp.int32, L)
  psum = jnp.zeros((L,), dtype=jnp.float32)
  for g in range(QPW // L):
    m = None
    for j in range(L):
      col = plsc.load_gather(acc_v, [lanes * L + (g * L * L + j)])
      m = col if m is None else jnp.minimum(m, col)
    psum = psum + m
  sv[...] = psum
  pltpu.sync_copy(sv, out_hbm.at[wid])


def kernel(query, ref):
  qrep = jnp.broadcast_to(query.T[:, :, None], (3, Q, L)).reshape(3, Q * L)
  rT = ref.T
  out = _chamfer_sc(qrep[0], qrep[1], qrep[2], rT[0], rT[1], rT[2])
  return jnp.sum(out) / jnp.float32(Q)


# speedup vs baseline: 8.9618x; 2.7991x over previous
import functools

import jax
import jax.numpy as jnp
from jax import lax
from jax.experimental import pallas as pl
from jax.experimental.pallas import tpu as pltpu
from jax.experimental.pallas import tpu_sc as plsc

NC = 2
NS = 16
L = 16
NW = NC * NS

Q = 2048
R = 32768
R_SC = 8192
R_TC = R - R_SC
QPW = Q // NW
QT = 4
NQT = QPW // QT
RV = R_SC // L
RU = 4
NRI = RV // RU

BR = 512
NB = R_TC // BR
KP = 16

_mesh = plsc.VectorSubcoreMesh(
    core_axis_name="c", subcore_axis_name="s", num_cores=NC, num_subcores=NS
)


def _round_bf16(v):
  u = lax.bitcast_convert_type(v, jnp.uint32)
  odd = lax.shift_right_logical(u, jnp.uint32(16)) & jnp.uint32(1)
  r = (u + jnp.uint32(0x7FFF) + odd) & jnp.uint32(0xFFFF0000)
  return lax.bitcast_convert_type(r, jnp.float32)


@functools.partial(
    pl.kernel,
    out_type=jax.ShapeDtypeStruct((NW, QPW), jnp.float32),
    mesh=_mesh,
    compiler_params=pltpu.CompilerParams(needs_layout_passes=False),
    scratch_types=[
        pltpu.VMEM((QPW * L,), jnp.float32),
        pltpu.VMEM((QPW * L,), jnp.float32),
        pltpu.VMEM((QPW * L,), jnp.float32),
        pltpu.VMEM((R_SC,), jnp.float32),
        pltpu.VMEM((R_SC,), jnp.float32),
        pltpu.VMEM((R_SC,), jnp.float32),
        pltpu.VMEM((R_SC,), jnp.float32),
        pltpu.VMEM((QPW * L,), jnp.float32),
        pltpu.VMEM((QPW,), jnp.float32),
    ],
)
def _chamfer_sc(qx_hbm, qy_hbm, qz_hbm, rx_hbm, ry_hbm, rz_hbm, out_hbm,
                qx_v, qy_v, qz_v, rx_v, ry_v, rz_v, rr_v, acc_v, sv):
  wid = lax.axis_index("c") * NS + lax.axis_index("s")
  qbase = wid * (QPW * L)

  pltpu.sync_copy(qx_hbm.at[pl.ds(qbase, QPW * L)], qx_v)
  pltpu.sync_copy(qy_hbm.at[pl.ds(qbase, QPW * L)], qy_v)
  pltpu.sync_copy(qz_hbm.at[pl.ds(qbase, QPW * L)], qz_v)
  pltpu.sync_copy(rx_hbm.at[pl.ds(0, R_SC)], rx_v)
  pltpu.sync_copy(ry_hbm.at[pl.ds(0, R_SC)], ry_v)
  pltpu.sync_copy(rz_hbm.at[pl.ds(0, R_SC)], rz_v)

  inf16 = jnp.full((L,), jnp.inf, dtype=jnp.float32)

  def prologue(j, carry):
    off = j * L
    x = rx_v[pl.ds(off, L)]
    y = ry_v[pl.ds(off, L)]
    z = rz_v[pl.ds(off, L)]
    rr_v[pl.ds(off, L)] = x * x + y * y + z * z
    rx_v[pl.ds(off, L)] = _round_bf16(x)
    ry_v[pl.ds(off, L)] = _round_bf16(y)
    rz_v[pl.ds(off, L)] = _round_bf16(z)
    return carry

  lax.fori_loop(0, RV, prologue, jnp.int32(0))

  def qtile_body(qt, carry):
    qq = []
    ax = []
    ay = []
    az = []
    for t in range(QT):
      off = (qt * QT + t) * L
      qxv = qx_v[pl.ds(off, L)]
      qyv = qy_v[pl.ds(off, L)]
      qzv = qz_v[pl.ds(off, L)]
      qq.append(qxv * qxv + qyv * qyv + qzv * qzv)
      ax.append(-2.0 * _round_bf16(qxv))
      ay.append(-2.0 * _round_bf16(qyv))
      az.append(-2.0 * _round_bf16(qzv))

    def rbody(i, accs):
      accs = list(accs)
      for u in range(RU):
        base = (i * RU + u) * L
        rxv = rx_v[pl.ds(base, L)]
        ryv = ry_v[pl.ds(base, L)]
        rzv = rz_v[pl.ds(base, L)]
        rrv = rr_v[pl.ds(base, L)]
        for t in range(QT):
          d = rrv + rxv * ax[t] + ryv * ay[t] + rzv * az[t]
          accs[t] = jnp.minimum(accs[t], d)
      return tuple(accs)

    accs = lax.fori_loop(0, NRI, rbody, (inf16,) * QT)
    for t in range(QT):
      soff = (qt * QT + t) * L
      acc_v[pl.ds(soff, L)] = accs[t] + qq[t]
    return carry

  lax.fori_loop(0, NQT, qtile_body, jnp.int32(0))

  lanes = lax.iota(jnp.int32, L)
  for g in range(QPW // L):
    m = None
    for j in range(L):
      col = plsc.load_gather(acc_v, [lanes * L + (g * L * L + j)])
      m = col if m is None else jnp.minimum(m, col)
    sv[pl.ds(g * L, L)] = m
  pltpu.sync_copy(sv, out_hbm.at[wid])


def _tc_body(qb_ref, rbt_ref, rf_ref, qf_ref, out_ref):
  j = pl.program_id(0)

  @pl.when(j == 0)
  def _():
    out_ref[...] = jnp.full_like(out_ref, jnp.inf)

  rf = rf_ref[...]
  rr = jnp.sum(rf * rf, axis=0, keepdims=True)
  rbt2 = rbt_ref[...] * jnp.bfloat16(-2.0)
  dots = jnp.dot(qb_ref[...], rbt2,
                 preferred_element_type=jnp.float32)
  m = jnp.min(dots + rr, axis=1, keepdims=True)
  out_ref[...] = jnp.minimum(out_ref[...], m)

  @pl.when(j == NB - 1)
  def _():
    qf = qf_ref[...]
    qq = jnp.sum(qf * qf, axis=1, keepdims=True)
    out_ref[...] = out_ref[...] + qq


_chamfer_tc = pl.pallas_call(
    _tc_body,
    grid=(NB,),
    in_specs=[
        pl.BlockSpec((Q, KP), lambda j: (0, 0)),
        pl.BlockSpec((KP, BR), lambda j: (0, j)),
        pl.BlockSpec((8, BR), lambda j: (0, j)),
        pl.BlockSpec((Q, 8), lambda j: (0, 0)),
    ],
    out_specs=pl.BlockSpec((Q, 1), lambda j: (0, 0)),
    out_shape=jax.ShapeDtypeStruct((Q, 1), jnp.float32),
    compiler_params=pltpu.CompilerParams(
        dimension_semantics=("arbitrary",),
    ),
)


def kernel(query, ref):
  qrep = jnp.broadcast_to(query.T[:, :, None], (3, Q, L)).reshape(3, Q * L)
  rT = ref.T

  ref_tc = ref[R_SC:]
  qb = jnp.zeros((Q, KP), jnp.bfloat16).at[:, :3].set(query.astype(jnp.bfloat16))
  rbt = jnp.zeros((KP, R_TC), jnp.bfloat16).at[:3, :].set(
      ref_tc.T.astype(jnp.bfloat16))
  rf = jnp.zeros((8, R_TC), jnp.float32).at[:3, :].set(ref_tc.T)
  qf = jnp.zeros((Q, 8), jnp.float32).at[:, :3].set(query)

  sc_mins = _chamfer_sc(qrep[0], qrep[1], qrep[2], rT[0], rT[1], rT[2])
  tc_mins = _chamfer_tc(qb, rbt, rf, qf)

  mins = jnp.minimum(sc_mins.reshape(Q), tc_mins.reshape(Q))
  return jnp.sum(mins) / jnp.float32(Q)
